# baseline (speedup 1.0000x reference)
import functools

import jax
import jax.numpy as jnp
from jax import lax
from jax.experimental import pallas as pl
from jax.experimental.pallas import tpu as pltpu
from jax.experimental.pallas import tpu_sc as plsc

NG = 10000
ND = 2000
DIN = 512
DH = 128
L = 2

NC = 2
NS = 16
CHUNK = 80
RB = 200
_MESH = dict(core_axis_name="c", subcore_axis_name="s", num_cores=NC,
             num_subcores=NS)


def _strided_row_copy(s, n_rblk, do_copy):
    n_rpass = -(-n_rblk // NS)
    for i in range(n_rpass):
        blk = s + NS * i

        @pl.when(blk < n_rblk)
        def _():
            do_copy(blk * RB)


_SEG_RELS = (("ppi", 320000, NG, 0), ("gsea", 320000, NG, 0),
             ("pcc", 320000, NG, 0), ("dti_dg", 64000, ND, 1),
             ("dds", 64000, ND, 1), ("dti_gd", 64000, ND, 0))


@functools.lru_cache(maxsize=None)
def _sc_seg_all():
    NW = NC * NS

    @functools.partial(
        pl.kernel,
        out_type=tuple(
            jax.ShapeDtypeStruct((NC, ndst, DH), jnp.float32)
            for _, _, ndst, _ in _SEG_RELS),
        mesh=plsc.VectorSubcoreMesh(**_MESH),
        scratch_types=[
            pltpu.VMEM((4 * CHUNK,), jnp.int32),
            pltpu.VMEM((320000 // (NW * CHUNK), CHUNK), jnp.int32),
            pltpu.VMEM((64000 // (NW * CHUNK), CHUNK), jnp.int32),
            pltpu.VMEM((2, CHUNK, DH), jnp.float32),
            pltpu.VMEM_SHARED((NG, DH), jnp.float32),
        ] + [pltpu.SemaphoreType.DMA] * 8,
    )
    def seg_all(*refs):
        xs = refs[:2]
        eds = refs[2:14]
        zeros_hbm = refs[14]
        outs = refs[15:21]
        sidx, didx_b, didx_s, rows, acc = refs[21:26]
        sems = refs[26:]
        gsA, gsB, ssA, ssB = sems[0:4]
        isem = sems[4:8]
        c = lax.axis_index("c")
        s = lax.axis_index("s")
        wid = c * NS + s

        def phase(x_hbm, src_hbm, dst_hbm, out_hbm, didx, E, NDST):
            EW = E // NW
            n_chunks = EW // CHUNK
            assert n_chunks % 4 == 1
            n_bodies = (n_chunks - 1) // 4
            n_rblk = NDST // RB
            pltpu.sync_copy(dst_hbm.at[wid], didx)
            _strided_row_copy(s, n_rblk, lambda r0: pltpu.sync_copy(
                zeros_hbm.at[pl.ds(r0, RB)], acc.at[pl.ds(r0, RB)]))
            plsc.subcore_barrier()
            ebase = wid * EW

            def slot(sl):
                return sidx.at[pl.ds(sl * CHUNK, CHUNK)]

            def fire_idx(x, sl, guard=True):
                def go():
                    pltpu.async_copy(
                        src_hbm.at[pl.ds(ebase + x * CHUNK, CHUNK)],
                        slot(sl), isem[sl])
                if guard is True:
                    go()
                else:
                    pl.when(guard)(go)

            def w_idx(sl, guard=True):
                def go():
                    pltpu.make_async_copy(
                        src_hbm.at[pl.ds(ebase, CHUNK)], slot(sl),
                        isem[sl]).wait()
                if guard is True:
                    go()
                else:
                    pl.when(guard)(go)

            def fire_gather(sl, buf, sem, guard=True):
                def go():
                    pltpu.async_copy(x_hbm.at[slot(sl)], rows.at[buf], sem)
                if guard is True:
                    go()
                else:
                    pl.when(guard)(go)

            def w_gather(buf, sem):
                pltpu.make_async_copy(x_hbm.at[slot(0)], rows.at[buf],
                                      sem).wait()

            def fire_scatter(x, buf, sem):
                pltpu.async_copy(rows.at[buf], acc.at[didx.at[x]], sem,
                                 add=True)

            def w_scatter(buf, sem):
                pltpu.make_async_copy(rows.at[buf], acc.at[didx.at[0]],
                                      sem).wait()

            A, B = 0, 1
            for k in range(4):
                fire_idx(k, k)
            w_idx(0)
            fire_gather(0, B, gsB)
            w_gather(B, gsB)
            fire_idx(4, 0)
            fire_scatter(0, B, ssB)
            w_idx(1)
            fire_gather(1, A, gsA)

            def body(h, carry):
                q = 1 + 4 * h
                w_idx(2)
                w_scatter(B, ssB)
                fire_gather(2, B, gsB)
                w_gather(A, gsA)
                fire_idx(q + 4, 1, q + 4 < n_chunks)
                fire_scatter(q, A, ssA)

                w_idx(3)
                w_scatter(A, ssA)
                fire_gather(3, A, gsA)
                w_gather(B, gsB)
                fire_idx(q + 5, 2, q + 5 < n_chunks)
                fire_scatter(q + 1, B, ssB)

                w_idx(0)
                w_scatter(B, ssB)
                fire_gather(0, B, gsB)
                w_gather(A, gsA)
                fire_idx(q + 6, 3, q + 6 < n_chunks)
                fire_scatter(q + 2, A, ssA)

                w_idx(1, q + 4 < n_chunks)
                w_scatter(A, ssA)
                fire_gather(1, A, gsA, q + 4 < n_chunks)
                w_gather(B, gsB)
                fire_idx(q + 7, 0, q + 7 < n_chunks)
                fire_scatter(q + 3, B, ssB)
                return carry

            lax.fori_loop(0, n_bodies, body, 0)
            w_scatter(B, ssB)
            plsc.subcore_barrier()
            _strided_row_copy(s, n_rblk, lambda r0: pltpu.sync_copy(
                acc.at[pl.ds(r0, RB)], out_hbm.at[c, pl.ds(r0, RB)]))

        for i, (_, E, ndst, xsel) in enumerate(_SEG_RELS):
            phase(xs[xsel], eds[2 * i], eds[2 * i + 1], outs[i],
                  didx_b if E == 320000 else didx_s, E, ndst)

    return seg_all


_COUNT_RELS = (("ppi", 320000, NG), ("gsea", 320000, NG), ("pcc", 320000, NG),
               ("dti_dg", 64000, ND), ("dds", 64000, ND),
               ("dti_gd", 64000, ND))


@functools.lru_cache(maxsize=None)
def _sc_count_all():
    NW = NC * NS

    @functools.partial(
        pl.kernel,
        out_type=tuple(
            jax.ShapeDtypeStruct((NC, ndst, DH), jnp.float32)
            for _, _, ndst in _COUNT_RELS),
        mesh=plsc.VectorSubcoreMesh(**_MESH),
        scratch_types=[
            pltpu.VMEM((320000 // (NW * CHUNK), CHUNK), jnp.int32),
            pltpu.VMEM((64000 // (NW * CHUNK), CHUNK), jnp.int32),
            pltpu.VMEM((CHUNK, DH), jnp.float32),
            pltpu.VMEM_SHARED((NG, DH), jnp.float32),
            pltpu.SemaphoreType.DMA,
            pltpu.SemaphoreType.DMA,
        ],
    )
    def count_all(*refs):
        d3s = refs[:6]
        zeros_hbm, ones_hbm = refs[6:8]
        outs = refs[8:14]
        didx_b, didx_s, ones_v, acc, ssem0, ssem1 = refs[14:]
        c = lax.axis_index("c")
        s = lax.axis_index("s")
        wid = c * NS + s
        pltpu.sync_copy(ones_hbm, ones_v)

        def phase(d3, out, didx, E, NDST):
            EW = E // NW
            n_chunks = EW // CHUNK
            n_pairs = (n_chunks - 1) // 2
            n_rblk = NDST // RB
            pltpu.sync_copy(d3.at[wid], didx)
            _strided_row_copy(s, n_rblk, lambda r0: pltpu.sync_copy(
                zeros_hbm.at[pl.ds(r0, RB)], acc.at[pl.ds(r0, RB)]))
            plsc.subcore_barrier()

            def scatter(j, sem):
                return pltpu.async_copy(ones_v, acc.at[didx.at[j]], sem,
                                        add=True)

            scatter(0, ssem0).wait()

            def body(g, carry):
                s0 = scatter(1 + 2 * g, ssem0)
                s1 = scatter(2 + 2 * g, ssem1)
                s0.wait()
                s1.wait()
                return carry

            lax.fori_loop(0, n_pairs, body, 0)
            plsc.subcore_barrier()
            _strided_row_copy(s, n_rblk, lambda r0: pltpu.sync_copy(
                acc.at[pl.ds(r0, RB)], out.at[c, pl.ds(r0, RB)]))

        for i, (_, E, ndst) in enumerate(_COUNT_RELS):
            phase(d3s[i], outs[i], didx_b if E == 320000 else didx_s,
                  E, ndst)

    return count_all


def _proj_body(x_ref, w_ref, b_ref, o_ref):
    y = jnp.dot(x_ref[...], w_ref[...], preferred_element_type=jnp.float32)
    o_ref[...] = y + b_ref[...]


def _project(x, w, b, block_rows):
    n = x.shape[0]
    return pl.pallas_call(
        _proj_body,
        grid=(n // block_rows,),
        in_specs=[
            pl.BlockSpec((block_rows, DIN), lambda i: (i, 0)),
            pl.BlockSpec((DIN, DH), lambda i: (0, 0)),
            pl.BlockSpec((1, DH), lambda i: (0, 0)),
        ],
        out_specs=pl.BlockSpec((block_rows, DH), lambda i: (i, 0)),
        out_shape=jax.ShapeDtypeStruct((n, DH), jnp.float32),
    )(x, w, b)


def _rel_term(p, c, w):
    ssum = p[0] + p[1]
    cnt = c[0, :, 0:1] + c[1, :, 0:1]
    agg = ssum * (1.0 / jnp.maximum(cnt, 1.0))
    return jnp.dot(agg, w, preferred_element_type=jnp.float32)


def _layer_body(T, n_dti):
    def body(*refs):
        p_refs = refs[:T]
        c_refs = refs[T:2 * T]
        x_ref, wn_ref, wr_ref, b_ref, o_ref = refs[2 * T:]
        wr_sum = jnp.sum(wr_ref[...], axis=0)
        acc = jnp.dot(x_ref[...], wr_sum, preferred_element_type=jnp.float32)
        acc = acc + jnp.sum(b_ref[...], axis=0)[None, :]
        for t in range(T - n_dti):
            acc = acc + _rel_term(p_refs[t][...], c_refs[t][...], wn_ref[t])
        for t in range(T - n_dti, T):
            term = _rel_term(p_refs[t][...], c_refs[t][...], wn_ref[t])
            mask = (pl.program_id(0) == 0).astype(jnp.float32)
            acc = acc + mask * term
        o_ref[...] = jnp.maximum(acc, 0.0)

    return body


def _layer(partials, counts, x, wn, wr, b, block_rows, n_dti=0):
    T = len(partials)
    n = x.shape[0]
    pc_specs = []
    for arr in list(partials) + list(counts):
        rows = arr.shape[1]
        if rows == n:
            pc_specs.append(
                pl.BlockSpec((NC, block_rows, DH), lambda i: (0, i, 0)))
        else:
            pc_specs.append(
                pl.BlockSpec((NC, rows, DH), lambda i: (0, 0, 0)))
    return pl.pallas_call(
        _layer_body(T, n_dti),
        grid=(n // block_rows,),
        in_specs=pc_specs
        + [
            pl.BlockSpec((block_rows, DH), lambda i: (i, 0)),
            pl.BlockSpec((T, DH, DH), lambda i: (0, 0, 0)),
            pl.BlockSpec((T, DH, DH), lambda i: (0, 0, 0)),
            pl.BlockSpec((T, DH), lambda i: (0, 0)),
        ],
        out_specs=pl.BlockSpec((block_rows, DH), lambda i: (i, 0)),
        out_shape=jax.ShapeDtypeStruct((n, DH), jnp.float32),
    )(*partials, *counts, x, wn, wr, b)


def kernel(x_gene, x_drug, edge_index_ppi, edge_index_gsea, edge_index_pcc,
           edge_index_dds, edge_index_dti_dg, edge_index_dti_gd,
           Wg, bg, Wd, bd, Wl, bl, Wr):
    zeros_g = jnp.zeros((NG, DH), jnp.float32)
    zeros_d = jnp.zeros((ND, DH), jnp.float32)
    ones_c = jnp.ones((CHUNK, DH), jnp.float32)

    xg = _project(x_gene, Wg, bg.reshape(1, DH), 2000)
    xd = _project(x_drug, Wd, bd.reshape(1, DH), 2000)

    def chunked_dst(e):
        nw = NC * NS
        return e[1].reshape(nw, e.shape[1] // (nw * CHUNK), CHUNK)

    rels = {}
    for name, e, ndst in [
            ("ppi", edge_index_ppi, NG), ("gsea", edge_index_gsea, NG),
            ("pcc", edge_index_pcc, NG), ("dds", edge_index_dds, ND),
            ("dti_dg", edge_index_dti_dg, ND),
            ("dti_gd", edge_index_dti_gd, ND)]:
        rels[name] = (e[0], chunked_dst(e), e.shape[1], ndst)

    c_ppi, c_gsea, c_pcc, c_dti_dg, c_dds, c_dti_gd = _sc_count_all()(
        *(rels[name][1] for name, _, _ in _COUNT_RELS), zeros_g, ones_c)

    seg_edges = []
    for name, _, _, _ in _SEG_RELS:
        seg_edges.extend(rels[name][:2])

    for l in range(L):
        (p_ppi, p_gsea, p_pcc, p_dti_dg, p_dds, p_dti_gd) = _sc_seg_all()(
            xg, xd, *seg_edges, zeros_g)

        wn_g = jnp.stack([Wl[l, 0], Wl[l, 1], Wl[l, 2], Wl[l, 4]])
        wr_g = jnp.stack([Wr[l, 0], Wr[l, 1], Wr[l, 2], Wr[l, 4]])
        b_g = jnp.stack([bl[l, 0], bl[l, 1], bl[l, 2], bl[l, 4]])
        wn_d = jnp.stack([Wl[l, 3], Wl[l, 5]])
        wr_d = jnp.stack([Wr[l, 3], Wr[l, 5]])
        b_d = jnp.stack([bl[l, 3], bl[l, 5]])

        xg = _layer([p_ppi, p_gsea, p_pcc, p_dti_dg],
                    [c_ppi, c_gsea, c_pcc, c_dti_dg],
                    xg, wn_g, wr_g, b_g, 2000, n_dti=1)
        xd = _layer([p_dds, p_dti_gd], [c_dds, c_dti_gd],
                    xd, wn_d, wr_d, b_d, 2000)

    return xg, xd


# speedup vs baseline: 1.0101x; 1.0101x over previous
import functools

import jax
import jax.numpy as jnp
from jax import lax
from jax.experimental import pallas as pl
from jax.experimental.pallas import tpu as pltpu
from jax.experimental.pallas import tpu_sc as plsc

NG = 10000
ND = 2000
DIN = 512
DH = 128
L = 2

NC = 2
NS = 16
CHUNK = 80
RB = 200
_MESH = dict(core_axis_name="c", subcore_axis_name="s", num_cores=NC,
             num_subcores=NS)


def _strided_row_copy(s, n_rblk, do_copy):
    n_rpass = -(-n_rblk // NS)
    for i in range(n_rpass):
        blk = s + NS * i

        @pl.when(blk < n_rblk)
        def _():
            do_copy(blk * RB)


@functools.lru_cache(maxsize=None)
def _sc_segment_sum(E, NSRC, NDST):
    NW = NC * NS
    EW = E // NW
    assert EW % CHUNK == 0
    n_chunks = EW // CHUNK
    assert n_chunks % 4 == 1
    n_bodies = (n_chunks - 1) // 4
    assert NDST % RB == 0
    n_rblk = NDST // RB

    @functools.partial(
        pl.kernel,
        out_type=jax.ShapeDtypeStruct((NC, NDST, DH), jnp.float32),
        mesh=plsc.VectorSubcoreMesh(**_MESH),
        scratch_types=[
            pltpu.VMEM((4 * CHUNK,), jnp.int32),
            pltpu.VMEM((n_chunks, CHUNK), jnp.int32),
            pltpu.VMEM((2, CHUNK, DH), jnp.float32),
            pltpu.VMEM_SHARED((NDST, DH), jnp.float32),
        ] + [pltpu.SemaphoreType.DMA] * 8,
    )
    def seg_sum(x_hbm, src_hbm, dst_hbm, zeros_hbm, out_hbm,
                sidx, didx, rows, acc, *sems):
        gsA, gsB, ssA, ssB = sems[0:4]
        isem = sems[4:8]
        c = lax.axis_index("c")
        s = lax.axis_index("s")
        wid = c * NS + s
        pltpu.sync_copy(dst_hbm.at[wid], didx)
        _strided_row_copy(s, n_rblk, lambda r0: pltpu.sync_copy(
            zeros_hbm.at[pl.ds(r0, RB)], acc.at[pl.ds(r0, RB)]))
        plsc.subcore_barrier()
        ebase = wid * EW

        def slot(sl):
            return sidx.at[pl.ds(sl * CHUNK, CHUNK)]

        def fire_idx(x, sl, guard=True):
            def go():
                pltpu.async_copy(src_hbm.at[pl.ds(ebase + x * CHUNK, CHUNK)],
                                 slot(sl), isem[sl])
            if guard is True:
                go()
            else:
                pl.when(guard)(go)

        def w_idx(sl, guard=True):
            def go():
                pltpu.make_async_copy(
                    src_hbm.at[pl.ds(ebase, CHUNK)], slot(sl),
                    isem[sl]).wait()
            if guard is True:
                go()
            else:
                pl.when(guard)(go)

        def fire_gather(sl, buf, sem, guard=True):
            def go():
                pltpu.async_copy(x_hbm.at[slot(sl)], rows.at[buf], sem)
            if guard is True:
                go()
            else:
                pl.when(guard)(go)

        def w_gather(buf, sem):
            pltpu.make_async_copy(x_hbm.at[slot(0)], rows.at[buf], sem).wait()

        def fire_scatter(x, buf, sem):
            pltpu.async_copy(rows.at[buf], acc.at[didx.at[x]], sem, add=True)

        def w_scatter(buf, sem):
            pltpu.make_async_copy(rows.at[buf], acc.at[didx.at[0]],
                                  sem).wait()

        A, B = 0, 1
        for k in range(4):
            fire_idx(k, k)
        w_idx(0)
        fire_gather(0, B, gsB)
        w_gather(B, gsB)
        fire_idx(4, 0)
        fire_scatter(0, B, ssB)
        w_idx(1)
        fire_gather(1, A, gsA)

        def body(h, carry):
            q = 1 + 4 * h
            w_idx(2)
            w_scatter(B, ssB)
            fire_gather(2, B, gsB)
            w_gather(A, gsA)
            fire_idx(q + 4, 1, q + 4 < n_chunks)
            fire_scatter(q, A, ssA)

            w_idx(3)
            w_scatter(A, ssA)
            fire_gather(3, A, gsA)
            w_gather(B, gsB)
            fire_idx(q + 5, 2, q + 5 < n_chunks)
            fire_scatter(q + 1, B, ssB)

            w_idx(0)
            w_scatter(B, ssB)
            fire_gather(0, B, gsB)
            w_gather(A, gsA)
            fire_idx(q + 6, 3, q + 6 < n_chunks)
            fire_scatter(q + 2, A, ssA)

            w_idx(1, q + 4 < n_chunks)
            w_scatter(A, ssA)
            fire_gather(1, A, gsA, q + 4 < n_chunks)
            w_gather(B, gsB)
            fire_idx(q + 7, 0, q + 7 < n_chunks)
            fire_scatter(q + 3, B, ssB)
            return carry

        lax.fori_loop(0, n_bodies, body, 0)
        w_scatter(B, ssB)
        plsc.subcore_barrier()
        _strided_row_copy(s, n_rblk, lambda r0: pltpu.sync_copy(
            acc.at[pl.ds(r0, RB)], out_hbm.at[c, pl.ds(r0, RB)]))

    return seg_sum


_COUNT_RELS = (("ppi", 320000, NG), ("gsea", 320000, NG), ("pcc", 320000, NG),
               ("dti_dg", 64000, ND), ("dds", 64000, ND),
               ("dti_gd", 64000, ND))


@functools.lru_cache(maxsize=None)
def _sc_count_all():
    NW = NC * NS

    @functools.partial(
        pl.kernel,
        out_type=tuple(
            jax.ShapeDtypeStruct((NC, ndst, DH), jnp.float32)
            for _, _, ndst in _COUNT_RELS),
        mesh=plsc.VectorSubcoreMesh(**_MESH),
        scratch_types=[
            pltpu.VMEM((320000 // (NW * CHUNK), CHUNK), jnp.int32),
            pltpu.VMEM((64000 // (NW * CHUNK), CHUNK), jnp.int32),
            pltpu.VMEM((CHUNK, DH), jnp.float32),
            pltpu.VMEM_SHARED((NG, DH), jnp.float32),
            pltpu.SemaphoreType.DMA,
            pltpu.SemaphoreType.DMA,
        ],
    )
    def count_all(*refs):
        d3s = refs[:6]
        zeros_hbm, ones_hbm = refs[6:8]
        outs = refs[8:14]
        didx_b, didx_s, ones_v, acc, ssem0, ssem1 = refs[14:]
        c = lax.axis_index("c")
        s = lax.axis_index("s")
        wid = c * NS + s
        pltpu.sync_copy(ones_hbm, ones_v)

        def phase(d3, out, didx, E, NDST):
            EW = E // NW
            n_chunks = EW // CHUNK
            n_pairs = (n_chunks - 1) // 2
            n_rblk = NDST // RB
            pltpu.sync_copy(d3.at[wid], didx)
            _strided_row_copy(s, n_rblk, lambda r0: pltpu.sync_copy(
                zeros_hbm.at[pl.ds(r0, RB)], acc.at[pl.ds(r0, RB)]))
            plsc.subcore_barrier()

            def scatter(j, sem):
                return pltpu.async_copy(ones_v, acc.at[didx.at[j]], sem,
                                        add=True)

            scatter(0, ssem0).wait()

            def body(g, carry):
                s0 = scatter(1 + 2 * g, ssem0)
                s1 = scatter(2 + 2 * g, ssem1)
                s0.wait()
                s1.wait()
                return carry

            lax.fori_loop(0, n_pairs, body, 0)
            plsc.subcore_barrier()
            _strided_row_copy(s, n_rblk, lambda r0: pltpu.sync_copy(
                acc.at[pl.ds(r0, RB)], out.at[c, pl.ds(r0, RB)]))

        for i, (_, E, ndst) in enumerate(_COUNT_RELS):
            phase(d3s[i], outs[i], didx_b if E == 320000 else didx_s,
                  E, ndst)

    return count_all


def _proj_body(x_ref, w_ref, b_ref, o_ref):
    y = jnp.dot(x_ref[...], w_ref[...], preferred_element_type=jnp.float32)
    o_ref[...] = y + b_ref[...]


def _project(x, w, b, block_rows):
    n = x.shape[0]
    return pl.pallas_call(
        _proj_body,
        grid=(n // block_rows,),
        in_specs=[
            pl.BlockSpec((block_rows, DIN), lambda i: (i, 0)),
            pl.BlockSpec((DIN, DH), lambda i: (0, 0)),
            pl.BlockSpec((1, DH), lambda i: (0, 0)),
        ],
        out_specs=pl.BlockSpec((block_rows, DH), lambda i: (i, 0)),
        out_shape=jax.ShapeDtypeStruct((n, DH), jnp.float32),
    )(x, w, b)


def _rel_term(p, c, w):
    ssum = p[0] + p[1]
    cnt = c[0, :, 0:1] + c[1, :, 0:1]
    agg = ssum * (1.0 / jnp.maximum(cnt, 1.0))
    return jnp.dot(agg, w, preferred_element_type=jnp.float32)


def _layer_body(T, n_dti):
    def body(*refs):
        p_refs = refs[:T]
        c_refs = refs[T:2 * T]
        x_ref, wn_ref, wr_ref, b_ref, o_ref = refs[2 * T:]
        wr_sum = jnp.sum(wr_ref[...], axis=0)
        acc = jnp.dot(x_ref[...], wr_sum, preferred_element_type=jnp.float32)
        acc = acc + jnp.sum(b_ref[...], axis=0)[None, :]
        for t in range(T - n_dti):
            acc = acc + _rel_term(p_refs[t][...], c_refs[t][...], wn_ref[t])
        for t in range(T - n_dti, T):
            term = _rel_term(p_refs[t][...], c_refs[t][...], wn_ref[t])
            mask = (pl.program_id(0) == 0).astype(jnp.float32)
            acc = acc + mask * term
        o_ref[...] = jnp.maximum(acc, 0.0)

    return body


def _layer(partials, counts, x, wn, wr, b, block_rows, n_dti=0):
    T = len(partials)
    n = x.shape[0]
    pc_specs = []
    for arr in list(partials) + list(counts):
        rows = arr.shape[1]
        if rows == n:
            pc_specs.append(
                pl.BlockSpec((NC, block_rows, DH), lambda i: (0, i, 0)))
        else:
            pc_specs.append(
                pl.BlockSpec((NC, rows, DH), lambda i: (0, 0, 0)))
    return pl.pallas_call(
        _layer_body(T, n_dti),
        grid=(n // block_rows,),
        in_specs=pc_specs
        + [
            pl.BlockSpec((block_rows, DH), lambda i: (i, 0)),
            pl.BlockSpec((T, DH, DH), lambda i: (0, 0, 0)),
            pl.BlockSpec((T, DH, DH), lambda i: (0, 0, 0)),
            pl.BlockSpec((T, DH), lambda i: (0, 0)),
        ],
        out_specs=pl.BlockSpec((block_rows, DH), lambda i: (i, 0)),
        out_shape=jax.ShapeDtypeStruct((n, DH), jnp.float32),
    )(*partials, *counts, x, wn, wr, b)


def kernel(x_gene, x_drug, edge_index_ppi, edge_index_gsea, edge_index_pcc,
           edge_index_dds, edge_index_dti_dg, edge_index_dti_gd,
           Wg, bg, Wd, bd, Wl, bl, Wr):
    zeros_g = jnp.zeros((NG, DH), jnp.float32)
    zeros_d = jnp.zeros((ND, DH), jnp.float32)
    ones_c = jnp.ones((CHUNK, DH), jnp.float32)

    xg = _project(x_gene, Wg, bg.reshape(1, DH), 2000)
    xd = _project(x_drug, Wd, bd.reshape(1, DH), 2000)

    def chunked_dst(e):
        nw = NC * NS
        return e[1].reshape(nw, e.shape[1] // (nw * CHUNK), CHUNK)

    rels = {}
    for name, e, ndst in [
            ("ppi", edge_index_ppi, NG), ("gsea", edge_index_gsea, NG),
            ("pcc", edge_index_pcc, NG), ("dds", edge_index_dds, ND),
            ("dti_dg", edge_index_dti_dg, ND),
            ("dti_gd", edge_index_dti_gd, ND)]:
        rels[name] = (e[0], chunked_dst(e), e.shape[1], ndst)

    c_ppi, c_gsea, c_pcc, c_dti_dg, c_dds, c_dti_gd = _sc_count_all()(
        *(rels[name][1] for name, _, _ in _COUNT_RELS), zeros_g, ones_c)

    def seg(name, x):
        sflat, d3, E, ndst = rels[name]
        zeros = zeros_g if ndst == NG else zeros_d
        return _sc_segment_sum(E, x.shape[0], ndst)(x, sflat, d3, zeros)

    for l in range(L):
        p_ppi = seg("ppi", xg)
        p_gsea = seg("gsea", xg)
        p_pcc = seg("pcc", xg)
        p_dti_dg = seg("dti_dg", xd)
        p_dds = seg("dds", xd)
        p_dti_gd = seg("dti_gd", xg)

        wn_g = jnp.stack([Wl[l, 0], Wl[l, 1], Wl[l, 2], Wl[l, 4]])
        wr_g = jnp.stack([Wr[l, 0], Wr[l, 1], Wr[l, 2], Wr[l, 4]])
        b_g = jnp.stack([bl[l, 0], bl[l, 1], bl[l, 2], bl[l, 4]])
        wn_d = jnp.stack([Wl[l, 3], Wl[l, 5]])
        wr_d = jnp.stack([Wr[l, 3], Wr[l, 5]])
        b_d = jnp.stack([bl[l, 3], bl[l, 5]])

        xg = _layer([p_ppi, p_gsea, p_pcc, p_dti_dg],
                    [c_ppi, c_gsea, c_pcc, c_dti_dg],
                    xg, wn_g, wr_g, b_g, 2000, n_dti=1)
        xd = _layer([p_dds, p_dti_gd], [c_dds, c_dti_gd],
                    xd, wn_d, wr_d, b_d, 2000)

    return xg, xd
